# baseline (speedup 1.0000x reference)
import functools

import jax
import jax.numpy as jnp
from jax import lax
from jax.experimental import pallas as pl
from jax.experimental.pallas import tpu as pltpu
from jax.experimental.pallas import tpu_sc as plsc

BATCH = 16384
N_FIELDS = 26
EMBED_DIM = 16

NUM_CORES = 2
NUM_SUBCORES = 16
NW = NUM_CORES * NUM_SUBCORES

B_TOTAL = BATCH * N_FIELDS
PER_W = B_TOTAL // NW
CHUNK = 128
N_CHUNK = PER_W // CHUNK
INNER = 13
OUTER = N_CHUNK // INNER
TILE_ROWS = INNER * CHUNK

_mesh = plsc.VectorSubcoreMesh(
    core_axis_name="c", subcore_axis_name="s",
    num_cores=NUM_CORES, num_subcores=NUM_SUBCORES,
)


@functools.partial(
    pl.kernel,
    out_type=jax.ShapeDtypeStruct((B_TOTAL, EMBED_DIM), jnp.float32),
    mesh=_mesh,
    compiler_params=pltpu.CompilerParams(use_tc_tiling_on_sc=False),
    scratch_types=[
        pltpu.VMEM((N_CHUNK, CHUNK), jnp.int32),
        pltpu.VMEM((TILE_ROWS, EMBED_DIM), jnp.float32),
        pltpu.SemaphoreType.DMA,
    ],
)
def _gather_rows(idx_hbm, table_hbm, out_hbm, idx_v, rows_v, sem):
    wid = lax.axis_index("s") * NUM_CORES + lax.axis_index("c")
    base_chunk = wid * N_CHUNK
    pltpu.sync_copy(idx_hbm.at[pl.ds(base_chunk, N_CHUNK)], idx_v)

    @pl.loop(0, OUTER)
    def _tile(g):
        copies = []
        for j in range(INNER):
            copies.append(
                pltpu.async_copy(
                    table_hbm.at[idx_v.at[g * INNER + j]],
                    rows_v.at[pl.ds(j * CHUNK, CHUNK)],
                    sem,
                )
            )
        for c in copies:
            c.wait()
        pltpu.sync_copy(
            rows_v,
            out_hbm.at[pl.ds((base_chunk + g * INNER) * CHUNK, TILE_ROWS)],
        )


def kernel(inputs, table):
    idx = inputs.reshape(B_TOTAL // CHUNK, CHUNK)
    rows = _gather_rows(idx, table)
    return rows.reshape(BATCH, N_FIELDS, EMBED_DIM)


# speedup vs baseline: 1.2706x; 1.2706x over previous
import functools

import jax
import jax.numpy as jnp
from jax import lax
from jax.experimental import pallas as pl
from jax.experimental.pallas import tpu as pltpu
from jax.experimental.pallas import tpu_sc as plsc

BATCH = 16384
N_FIELDS = 26
EMBED_DIM = 16

NUM_CORES = 2
NUM_SUBCORES = 16
NW = NUM_CORES * NUM_SUBCORES

B_PER_W = BATCH // NW
B_PER_TILE = 64
N_TILES = B_PER_W // B_PER_TILE
FIRE = 16

_mesh = plsc.VectorSubcoreMesh(
    core_axis_name="c", subcore_axis_name="s",
    num_cores=NUM_CORES, num_subcores=NUM_SUBCORES,
)


@functools.partial(
    pl.kernel,
    out_type=jax.ShapeDtypeStruct((BATCH, N_FIELDS, EMBED_DIM), jnp.float32),
    mesh=_mesh,
    compiler_params=pltpu.CompilerParams(use_tc_tiling_on_sc=False),
    scratch_types=[
        pltpu.VMEM((B_PER_W, N_FIELDS), jnp.int32),
        pltpu.VMEM((B_PER_TILE, N_FIELDS, EMBED_DIM), jnp.float32),
        pltpu.SemaphoreType.DMA,
    ],
)
def _gather_rows(idx_hbm, table_hbm, out_hbm, idx_v, rows_v, sem):
    wid = lax.axis_index("s") * NUM_CORES + lax.axis_index("c")
    pltpu.sync_copy(idx_hbm.at[pl.ds(wid * B_PER_W, B_PER_W)], idx_v)

    @pl.loop(0, N_TILES)
    def _tile(g):
        for j0 in range(0, B_PER_TILE, FIRE):
            copies = []
            for j in range(j0, j0 + FIRE):
                copies.append(
                    pltpu.async_copy(
                        table_hbm.at[idx_v.at[g * B_PER_TILE + j]],
                        rows_v.at[j],
                        sem,
                    )
                )
            for c in copies:
                c.wait()
        pltpu.sync_copy(
            rows_v,
            out_hbm.at[pl.ds(wid * B_PER_W + g * B_PER_TILE, B_PER_TILE)],
        )


def kernel(inputs, table):
    return _gather_rows(inputs, table)


# speedup vs baseline: 1.4305x; 1.1258x over previous
import functools

import jax
import jax.numpy as jnp
from jax import lax
from jax.experimental import pallas as pl
from jax.experimental.pallas import tpu as pltpu
from jax.experimental.pallas import tpu_sc as plsc

BATCH = 16384
N_FIELDS = 26
EMBED_DIM = 16

NUM_CORES = 2
NUM_SUBCORES = 16
NW = NUM_CORES * NUM_SUBCORES

B_PER_W = BATCH // NW
BT = 128
K_BLOCKS = B_PER_W // BT
FIRE = 16
ROWS_PER_BLOCK = BT * N_FIELDS

_mesh = plsc.VectorSubcoreMesh(
    core_axis_name="c", subcore_axis_name="s",
    num_cores=NUM_CORES, num_subcores=NUM_SUBCORES,
)


@functools.partial(
    pl.kernel,
    out_type=jax.ShapeDtypeStruct(
        (N_FIELDS, EMBED_DIM // 8, BATCH // BT, 8, BT), jnp.float32),
    mesh=_mesh,
    compiler_params=pltpu.CompilerParams(
        use_tc_tiling_on_sc=False, needs_layout_passes=False),
    scratch_types=[
        pltpu.VMEM((B_PER_W, N_FIELDS), jnp.int32),
        pltpu.VMEM((ROWS_PER_BLOCK, EMBED_DIM), jnp.float32),
        pltpu.VMEM((N_FIELDS, EMBED_DIM // 8, 8, BT), jnp.float32),
        pltpu.SemaphoreType.DMA,
    ],
)
def _gather_rows(idx_hbm, table_hbm, out_hbm, idx_v, rows_v, stage_v, sem):
    wid = lax.axis_index("s") * NUM_CORES + lax.axis_index("c")
    pltpu.sync_copy(idx_hbm.at[pl.ds(wid * B_PER_W, B_PER_W)], idx_v)
    lane = jnp.arange(16, dtype=jnp.int32)
    lane26 = lane * N_FIELDS

    @pl.loop(0, K_BLOCKS)
    def _block(k):
        for s0 in range(0, BT, FIRE):
            copies = []
            for t in range(FIRE):
                bl = s0 + t
                copies.append(
                    pltpu.async_copy(
                        table_hbm.at[idx_v.at[k * BT + bl]],
                        rows_v.at[pl.ds(bl * N_FIELDS, N_FIELDS)],
                        sem,
                    )
                )
            for c in copies:
                c.wait()

        @pl.loop(0, N_FIELDS)
        def _field(f):
            for e in range(EMBED_DIM):
                et, es = divmod(e, 8)
                col = lane * 0 + e
                for h in range(BT // 16):
                    row_ids = lane26 + (f + h * 16 * N_FIELDS)
                    v = plsc.load_gather(rows_v, [row_ids, col])
                    stage_v[f, et, es, pl.ds(h * 16, 16)] = v

        pltpu.sync_copy(stage_v, out_hbm.at[:, :, wid * K_BLOCKS + k])


def kernel(inputs, table):
    out5 = _gather_rows(inputs, table)
    return out5.transpose(2, 4, 0, 1, 3).reshape(BATCH, N_FIELDS, EMBED_DIM)


# speedup vs baseline: 1.6507x; 1.1539x over previous
import functools

import jax
import jax.numpy as jnp
from jax import lax
from jax.experimental import pallas as pl
from jax.experimental.pallas import tpu as pltpu
from jax.experimental.pallas import tpu_sc as plsc

BATCH = 16384
N_FIELDS = 26
EMBED_DIM = 16

NUM_CORES = 2
NUM_SUBCORES = 16
NW = NUM_CORES * NUM_SUBCORES

B_PER_W = BATCH // NW
BT = 128
K_BLOCKS = B_PER_W // BT
ROWS_PER_BLOCK = BT * N_FIELDS
STREAM = 128
S_PER_BLOCK = ROWS_PER_BLOCK // STREAM
FIRE = 13
IDX_ROWS = K_BLOCKS * S_PER_BLOCK

_mesh = plsc.VectorSubcoreMesh(
    core_axis_name="c", subcore_axis_name="s",
    num_cores=NUM_CORES, num_subcores=NUM_SUBCORES,
)


@functools.partial(
    pl.kernel,
    out_type=jax.ShapeDtypeStruct(
        (N_FIELDS, EMBED_DIM // 8, BATCH // BT, 8, BT), jnp.float32),
    mesh=_mesh,
    compiler_params=pltpu.CompilerParams(
        use_tc_tiling_on_sc=False, needs_layout_passes=False),
    scratch_types=[
        pltpu.VMEM((IDX_ROWS, STREAM), jnp.int32),
        pltpu.VMEM((ROWS_PER_BLOCK, EMBED_DIM), jnp.float32),
        pltpu.VMEM((N_FIELDS, EMBED_DIM // 8, 8, BT), jnp.float32),
        pltpu.SemaphoreType.DMA,
    ],
)
def _gather_rows(idx_hbm, table_hbm, out_hbm, idx_v, rows_v, stage_v, sem):
    wid = lax.axis_index("s") * NUM_CORES + lax.axis_index("c")
    pltpu.sync_copy(idx_hbm.at[pl.ds(wid * IDX_ROWS, IDX_ROWS)], idx_v)
    lane = jnp.arange(16, dtype=jnp.int32)
    lane26 = [lane * N_FIELDS + h * 16 * N_FIELDS for h in range(BT // 16)]

    @pl.loop(0, K_BLOCKS)
    def _block(k):
        for s0 in range(0, S_PER_BLOCK, FIRE):
            copies = []
            for t in range(FIRE):
                s = s0 + t
                copies.append(
                    pltpu.async_copy(
                        table_hbm.at[idx_v.at[k * S_PER_BLOCK + s]],
                        rows_v.at[pl.ds(s * STREAM, STREAM)],
                        sem,
                    )
                )
            for c in copies:
                c.wait()

        @plsc.parallel_loop(0, N_FIELDS, unroll=2)
        def _field(f):
            for e in range(EMBED_DIM):
                et, es = divmod(e, 8)
                col = lane * 0 + e
                for h in range(BT // 16):
                    v = plsc.load_gather(rows_v, [lane26[h] + f, col])
                    stage_v[f, et, es, pl.ds(h * 16, 16)] = v

        pltpu.sync_copy(stage_v, out_hbm.at[:, :, wid * K_BLOCKS + k])


def kernel(inputs, table):
    idx = inputs.reshape(BATCH * N_FIELDS // STREAM, STREAM)
    out5 = _gather_rows(idx, table)
    return out5.transpose(2, 4, 0, 1, 3).reshape(BATCH, N_FIELDS, EMBED_DIM)
